# Initial kernel scaffold; baseline (speedup 1.0000x reference)
import jax
import jax.numpy as jnp
from jax.experimental import pallas as pl


def kernel(x, table):
    raise NotImplementedError("write your pallas kernel here")


# speedup vs baseline: 4.9830x; 4.9830x over previous
import functools

import jax
import jax.numpy as jnp
from jax import lax
from jax.experimental import pallas as pl
from jax.experimental.pallas import tpu as pltpu
from jax.experimental.pallas import tpu_sc as plsc

EMBED_DIM = 64
NUM_WORKERS = 32
GROUP = 1024
CHUNK = 128
K = GROUP // CHUNK


def _gather_rows(table, idx_flat):
    B = idx_flat.shape[0]
    b_per_w = B // NUM_WORKERS
    num_groups = b_per_w // GROUP

    @functools.partial(
        pl.kernel,
        out_type=jax.ShapeDtypeStruct((B, EMBED_DIM), jnp.float32),
        mesh=plsc.VectorSubcoreMesh(
            core_axis_name="c", subcore_axis_name="s"
        ),
        scratch_types=[
            pltpu.VMEM((GROUP,), jnp.int32),
            pltpu.VMEM((GROUP, EMBED_DIM), jnp.float32),
            pltpu.SemaphoreType.DMA,
        ],
        compiler_params=pltpu.CompilerParams(use_tc_tiling_on_sc=False),
    )
    def k(table_hbm, idx_hbm, out_hbm, idx_v, rows_v, sem):
        wid = lax.axis_index("s") * 2 + lax.axis_index("c")
        base = wid * b_per_w

        def body(g, _):
            gbase = base + g * GROUP
            pltpu.sync_copy(idx_hbm.at[pl.ds(gbase, GROUP)], idx_v)
            descs = []
            for j in range(K):
                descs.append(
                    pltpu.async_copy(
                        table_hbm.at[idx_v.at[pl.ds(j * CHUNK, CHUNK)]],
                        rows_v.at[pl.ds(j * CHUNK, CHUNK)],
                        sem,
                    )
                )
            for d in descs:
                d.wait()
            pltpu.sync_copy(rows_v, out_hbm.at[pl.ds(gbase, GROUP)])
            return 0

        lax.fori_loop(0, num_groups, body, 0)

    return k(table, idx_flat)


def kernel(x, table):
    idx_flat = x.reshape(-1).astype(jnp.int32)
    out = _gather_rows(table, idx_flat)
    return out.reshape(x.shape + (EMBED_DIM,))


# speedup vs baseline: 5.0835x; 1.0202x over previous
import functools

import jax
import jax.numpy as jnp
from jax import lax
from jax.experimental import pallas as pl
from jax.experimental.pallas import tpu as pltpu
from jax.experimental.pallas import tpu_sc as plsc

EMBED_DIM = 64
NUM_WORKERS = 32
GROUP = 640
CHUNK = 128
K = GROUP // CHUNK


def _gather_rows(table, idx_flat):
    B = idx_flat.shape[0]
    b_per_w = B // NUM_WORKERS
    num_groups = b_per_w // GROUP
    num_pairs = num_groups // 2

    @functools.partial(
        pl.kernel,
        out_type=jax.ShapeDtypeStruct((B, EMBED_DIM), jnp.float32),
        mesh=plsc.VectorSubcoreMesh(
            core_axis_name="c", subcore_axis_name="s"
        ),
        scratch_types=[
            pltpu.VMEM((GROUP,), jnp.int32),
            pltpu.VMEM((GROUP,), jnp.int32),
            pltpu.VMEM((GROUP, EMBED_DIM), jnp.float32),
            pltpu.VMEM((GROUP, EMBED_DIM), jnp.float32),
            pltpu.SemaphoreType.DMA,
            pltpu.SemaphoreType.DMA,
            pltpu.SemaphoreType.DMA,
            pltpu.SemaphoreType.DMA,
        ],
        compiler_params=pltpu.CompilerParams(use_tc_tiling_on_sc=False),
    )
    def k(table_hbm, idx_hbm, out_hbm,
          idx0, idx1, rows0, rows1, gsem0, gsem1, ssem0, ssem1):
        wid = lax.axis_index("s") * 2 + lax.axis_index("c")
        base = wid * b_per_w

        def fire_gathers(idx_v, rows_v, sem, gbase):
            pltpu.sync_copy(idx_hbm.at[pl.ds(gbase, GROUP)], idx_v)
            for j in range(K):
                pltpu.async_copy(
                    table_hbm.at[idx_v.at[pl.ds(j * CHUNK, CHUNK)]],
                    rows_v.at[pl.ds(j * CHUNK, CHUNK)],
                    sem,
                )

        def drain_gathers(idx_v, rows_v, sem):
            for j in range(K):
                pltpu.make_async_copy(
                    table_hbm.at[idx_v.at[pl.ds(j * CHUNK, CHUNK)]],
                    rows_v.at[pl.ds(j * CHUNK, CHUNK)],
                    sem,
                ).wait()

        def store_wait(rows_v, sem, gbase):
            pltpu.make_async_copy(
                rows_v, out_hbm.at[pl.ds(gbase, GROUP)], sem
            ).wait()

        fire_gathers(idx0, rows0, gsem0, base)

        def body(h, _):
            g0 = base + (2 * h) * GROUP
            g1 = g0 + GROUP
            g2 = g1 + GROUP

            @pl.when(h > 0)
            def _():
                store_wait(rows1, ssem1, g1 - 2 * GROUP)

            fire_gathers(idx1, rows1, gsem1, g1)

            drain_gathers(idx0, rows0, gsem0)
            pltpu.async_copy(rows0, out_hbm.at[pl.ds(g0, GROUP)], ssem0)

            @pl.when(h < num_pairs - 1)
            def _():
                store_wait(rows0, ssem0, g0)
                fire_gathers(idx0, rows0, gsem0, g2)

            drain_gathers(idx1, rows1, gsem1)
            pltpu.async_copy(rows1, out_hbm.at[pl.ds(g1, GROUP)], ssem1)
            return 0

        lax.fori_loop(0, num_pairs, body, 0)

        last = base + (num_groups - 2) * GROUP
        store_wait(rows0, ssem0, last)
        store_wait(rows1, ssem1, last + GROUP)

    return k(table, idx_flat)


def kernel(x, table):
    idx_flat = x.reshape(-1).astype(jnp.int32)
    out = _gather_rows(table, idx_flat)
    return out.reshape(x.shape + (EMBED_DIM,))


# speedup vs baseline: 9.6779x; 1.9038x over previous
import functools

import jax
import jax.numpy as jnp
from jax import lax
from jax.experimental import pallas as pl
from jax.experimental.pallas import tpu as pltpu
from jax.experimental.pallas import tpu_sc as plsc

BATCH = 16384
HIST = 200
EMBED_DIM = 64
PADDED_DIM = 128
NUM_WORKERS = 32
GB = 4
GROUP = GB * HIST
CHUNKS = [(i * 128, 128) for i in range(GROUP // 128)]
if GROUP % 128:
    CHUNKS.append((GROUP - GROUP % 128, GROUP % 128))


def _gather_rows(table, idx):
    batches_per_w = BATCH // NUM_WORKERS
    num_groups = batches_per_w // GB
    num_pairs = num_groups // 2

    @functools.partial(
        pl.kernel,
        out_type=jax.ShapeDtypeStruct((BATCH, HIST, PADDED_DIM), jnp.float32),
        mesh=plsc.VectorSubcoreMesh(
            core_axis_name="c", subcore_axis_name="s"
        ),
        scratch_types=[
            pltpu.VMEM((GROUP,), jnp.int32),
            pltpu.VMEM((GROUP,), jnp.int32),
            pltpu.VMEM((GROUP, EMBED_DIM), jnp.float32),
            pltpu.VMEM((GROUP, EMBED_DIM), jnp.float32),
            pltpu.SemaphoreType.DMA,
            pltpu.SemaphoreType.DMA,
            pltpu.SemaphoreType.DMA,
            pltpu.SemaphoreType.DMA,
        ],
        compiler_params=pltpu.CompilerParams(use_tc_tiling_on_sc=False),
    )
    def k(table_hbm, idx_hbm, out_hbm,
          idx0, idx1, rows0, rows1, gsem0, gsem1, ssem0, ssem1):
        wid = lax.axis_index("s") * 2 + lax.axis_index("c")
        base = wid * batches_per_w

        def fire_gathers(idx_v, rows_v, sem, gbatch):
            pltpu.sync_copy(idx_hbm.at[pl.ds(gbatch * HIST, GROUP)], idx_v)
            for r, n in CHUNKS:
                pltpu.async_copy(
                    table_hbm.at[idx_v.at[pl.ds(r, n)]],
                    rows_v.at[pl.ds(r, n)],
                    sem,
                )

        def drain_gathers(idx_v, rows_v, sem):
            for r, n in CHUNKS:
                pltpu.make_async_copy(
                    table_hbm.at[idx_v.at[pl.ds(r, n)]],
                    rows_v.at[pl.ds(r, n)],
                    sem,
                ).wait()

        def fire_stores(rows_v, sem, gbatch):
            for b in range(GB):
                pltpu.async_copy(
                    rows_v.at[pl.ds(b * HIST, HIST)],
                    out_hbm.at[gbatch + b, pl.ds(0, HIST), pl.ds(0, EMBED_DIM)],
                    sem,
                )

        def store_wait(rows_v, sem, gbatch):
            for b in range(GB):
                pltpu.make_async_copy(
                    rows_v.at[pl.ds(b * HIST, HIST)],
                    out_hbm.at[gbatch + b, pl.ds(0, HIST), pl.ds(0, EMBED_DIM)],
                    sem,
                ).wait()

        fire_gathers(idx0, rows0, gsem0, base)

        def body(h, _):
            g0 = base + (2 * h) * GB
            g1 = g0 + GB
            g2 = g1 + GB

            @pl.when(h > 0)
            def _():
                store_wait(rows1, ssem1, g1 - 2 * GB)

            fire_gathers(idx1, rows1, gsem1, g1)

            drain_gathers(idx0, rows0, gsem0)
            fire_stores(rows0, ssem0, g0)

            @pl.when(h < num_pairs - 1)
            def _():
                store_wait(rows0, ssem0, g0)
                fire_gathers(idx0, rows0, gsem0, g2)

            drain_gathers(idx1, rows1, gsem1)
            fire_stores(rows1, ssem1, g1)
            return 0

        lax.fori_loop(0, num_pairs, body, 0)

        last = base + (num_groups - 2) * GB
        store_wait(rows0, ssem0, last)
        store_wait(rows1, ssem1, last + GB)

    return k(table, idx)


def kernel(x, table):
    idx_flat = x.reshape(-1).astype(jnp.int32)
    out = _gather_rows(table, idx_flat)
    return out[:, :, :EMBED_DIM]
